# baseline (speedup 1.0000x reference)
import functools
import jax
import jax.numpy as jnp
from jax import lax
from jax.experimental import pallas as pl
from jax.experimental.pallas import tpu as pltpu
from jax.experimental.pallas import tpu_sc as plsc

_B, _N, _M, _C1, _C2 = 8, 4096, 1024, 128, 256
_NBLK = 512
_BIG = 3.0e38

_NW = 32
_PW = _B * _N // _NW
_CP = 64
_LG = _C2 // 16


def _nn_body(u_ref, kt_ref, gidx_ref, wts_ref):
    b = pl.program_id(0)
    u = u_ref[0]
    kt = kt_ref[0]

    d2 = jnp.zeros((_NBLK, _M), jnp.float32)
    for d in range(3):
        diff = u[:, d:d + 1] - kt[d:d + 1, :]
        d2 = d2 + diff * diff

    ids = lax.broadcasted_iota(jnp.int32, (_NBLK, _M), 1)
    cur = d2
    mins = []
    idxs = []
    for _ in range(3):
        m = jnp.min(cur, axis=1, keepdims=True)
        i = jnp.min(jnp.where(cur == m, ids, _M), axis=1, keepdims=True)
        mins.append(m)
        idxs.append(i)
        cur = jnp.where(ids == i, _BIG, cur)

    r1 = 1.0 / (mins[0] + 1e-8)
    r2 = 1.0 / (mins[1] + 1e-8)
    r3 = 1.0 / (mins[2] + 1e-8)
    norm = r1 + r2 + r3

    gidx_ref[0] = jnp.concatenate(idxs, axis=1) + b * _M
    wts_ref[0] = jnp.concatenate(
        [jnp.broadcast_to(r1 / norm, (_NBLK, 16)),
         jnp.broadcast_to(r2 / norm, (_NBLK, 16)),
         jnp.broadcast_to(r3 / norm, (_NBLK, 16))], axis=1)


def _three_nn(unknown, kt):
    return pl.pallas_call(
        _nn_body,
        grid=(_B, _N // _NBLK),
        in_specs=[
            pl.BlockSpec((1, _NBLK, 3), lambda b, n: (b, n, 0)),
            pl.BlockSpec((1, 3, _M), lambda b, n: (b, 0, 0)),
        ],
        out_specs=[
            pl.BlockSpec((1, _NBLK, 3), lambda b, n: (b, n, 0)),
            pl.BlockSpec((1, _NBLK, 48), lambda b, n: (b, n, 0)),
        ],
        out_shape=[
            jax.ShapeDtypeStruct((_B, _N, 3), jnp.int32),
            jax.ShapeDtypeStruct((_B, _N, 48), jnp.float32),
        ],
    )(unknown, kt)


def _sc_interp_body(kft_hbm, gidx_hbm, wts_hbm, out_hbm,
                    idx_v, w_v, rows_v, acc_v, sem):
    wid = lax.axis_index("s") * 2 + lax.axis_index("c")

    def chunk_body(ci, carry):
        base = wid * _PW * 3 + ci * _CP * 3
        pltpu.sync_copy(gidx_hbm.at[pl.ds(base, 3 * _CP)], idx_v)
        pltpu.sync_copy(wts_hbm.at[pl.ds(base, 3 * _CP)], w_v)
        pltpu.async_copy(kft_hbm.at[idx_v], rows_v, sem).wait()

        def point_body(p, carry2):
            w0 = w_v[3 * p]
            w1 = w_v[3 * p + 1]
            w2 = w_v[3 * p + 2]
            for c in range(_LG):
                sl = pl.ds(16 * c, 16)
                acc = (w0 * rows_v[3 * p, sl]
                       + w1 * rows_v[3 * p + 1, sl]
                       + w2 * rows_v[3 * p + 2, sl])
                acc_v[p, sl] = acc
            return carry2

        lax.fori_loop(0, _CP, point_body, 0)
        pltpu.sync_copy(acc_v, out_hbm.at[pl.ds(wid * _PW + ci * _CP, _CP)])
        return carry

    lax.fori_loop(0, _PW // _CP, chunk_body, 0)


def _sc_interp(kft2, gidx_f, wts_f):
    mesh = plsc.VectorSubcoreMesh(core_axis_name="c", subcore_axis_name="s")
    f = functools.partial(
        pl.kernel,
        out_type=jax.ShapeDtypeStruct((_B * _N, _C2), jnp.float32),
        mesh=mesh,
        scratch_types=[
            pltpu.VMEM((3 * _CP,), jnp.int32),
            pltpu.VMEM((3 * _CP, 16), jnp.float32),
            pltpu.VMEM((3 * _CP, _C2), jnp.float32),
            pltpu.VMEM((_CP, _C2), jnp.float32),
            pltpu.SemaphoreType.DMA,
        ],
    )(_sc_interp_body)
    return f(kft2, gidx_f, wts_f)


def _mlp_body(interp_ref, uft_ref, w1_ref, w2_ref, out_ref):
    x = jnp.concatenate([interp_ref[0], uft_ref[0]], axis=1)
    h = jnp.maximum(jnp.dot(x, w1_ref[...],
                            preferred_element_type=jnp.float32), 0.0)
    h = jnp.maximum(jnp.dot(h, w2_ref[...],
                            preferred_element_type=jnp.float32), 0.0)
    out_ref[0] = h.T


def _mlp(interp3, uft, W1, W2):
    return pl.pallas_call(
        _mlp_body,
        grid=(_B, _N // _NBLK),
        in_specs=[
            pl.BlockSpec((1, _NBLK, _C2), lambda b, n: (b, n, 0)),
            pl.BlockSpec((1, _NBLK, _C1), lambda b, n: (b, n, 0)),
            pl.BlockSpec((_C1 + _C2, 256), lambda b, n: (0, 0)),
            pl.BlockSpec((256, 256), lambda b, n: (0, 0)),
        ],
        out_specs=pl.BlockSpec((1, 256, _NBLK), lambda b, n: (b, 0, n)),
        out_shape=jax.ShapeDtypeStruct((_B, 256, _N), jnp.float32),
    )(interp3, uft, W1, W2)


@jax.jit
def kernel(unknown, known, unknow_feats, known_feats, W1, W2):
    kt = known.swapaxes(1, 2)
    uft = unknow_feats.swapaxes(1, 2)
    kft2 = known_feats.swapaxes(1, 2).reshape(_B * _M, _C2)

    gidx, wts = _three_nn(unknown, kt)
    interp = _sc_interp(kft2, gidx.reshape(-1),
                        wts.reshape(_B * _N * 3, 16))
    return _mlp(interp.reshape(_B, _N, _C2), uft, W1, W2)


# speedup vs baseline: 1.2738x; 1.2738x over previous
import functools
import jax
import jax.numpy as jnp
from jax import lax
from jax.experimental import pallas as pl
from jax.experimental.pallas import tpu as pltpu
from jax.experimental.pallas import tpu_sc as plsc

_B, _N, _M, _C1, _C2 = 8, 4096, 1024, 128, 256
_NBLK = 512
_BIG = 3.0e38

_NW = 32
_PW = _B * _N // _NW
_CP = 128
_WPB = _N // _PW


def _nn_body(u_ref, kt_ref, i1_ref, i2_ref, i3_ref, w_ref):
    b = pl.program_id(0)
    u = u_ref[0]
    kt = kt_ref[0]

    d2 = jnp.zeros((_NBLK, _M), jnp.float32)
    for d in range(3):
        diff = u[:, d:d + 1] - kt[d:d + 1, :]
        d2 = d2 + diff * diff

    ids = lax.broadcasted_iota(jnp.int32, (_NBLK, _M), 1)
    cur = d2
    mins = []
    idxs = []
    for _ in range(3):
        m = jnp.min(cur, axis=1, keepdims=True)
        i = jnp.min(jnp.where(cur == m, ids, _M), axis=1, keepdims=True)
        mins.append(m)
        idxs.append(i)
        cur = jnp.where(ids == i, _BIG, cur)

    r1 = 1.0 / (mins[0] + 1e-8)
    r2 = 1.0 / (mins[1] + 1e-8)
    r3 = 1.0 / (mins[2] + 1e-8)
    norm = r1 + r2 + r3

    i1_ref[0, 0] = jnp.reshape(idxs[0] + b * _M, (_NBLK,))
    i2_ref[0, 0] = jnp.reshape(idxs[1] + b * _M, (_NBLK,))
    i3_ref[0, 0] = jnp.reshape(idxs[2] + b * _M, (_NBLK,))
    w_ref[0] = jnp.concatenate([r1 / norm, r2 / norm, r3 / norm], axis=1)


def _three_nn(unknown, kt):
    return pl.pallas_call(
        _nn_body,
        grid=(_B, _N // _NBLK),
        in_specs=[
            pl.BlockSpec((1, _NBLK, 3), lambda b, n: (b, n, 0)),
            pl.BlockSpec((1, 3, _M), lambda b, n: (b, 0, 0)),
        ],
        out_specs=[
            pl.BlockSpec((1, 1, _NBLK), lambda b, n: (b, 0, n)),
            pl.BlockSpec((1, 1, _NBLK), lambda b, n: (b, 0, n)),
            pl.BlockSpec((1, 1, _NBLK), lambda b, n: (b, 0, n)),
            pl.BlockSpec((1, _NBLK, 3), lambda b, n: (b, n, 0)),
        ],
        out_shape=[
            jax.ShapeDtypeStruct((_B, 1, _N), jnp.int32),
            jax.ShapeDtypeStruct((_B, 1, _N), jnp.int32),
            jax.ShapeDtypeStruct((_B, 1, _N), jnp.int32),
            jax.ShapeDtypeStruct((_B, _N, 3), jnp.float32),
        ],
    )(unknown, kt)


def _sc_gather_body(kft_hbm, i1_hbm, i2_hbm, i3_hbm,
                    r1_hbm, r2_hbm, r3_hbm,
                    idx_v, rows_v, sem):
    wid = lax.axis_index("s") * 2 + lax.axis_index("c")
    bi = wid // _WPB
    col0 = (wid % _WPB) * _PW

    def chunk_body(ci, carry):
        col = col0 + ci * _CP
        row0 = bi * _N + col
        for k, (ip, rp) in enumerate(((i1_hbm, r1_hbm), (i2_hbm, r2_hbm),
                                      (i3_hbm, r3_hbm))):
            pltpu.sync_copy(ip.at[bi, 0, pl.ds(col, _CP)], idx_v.at[k])
            pltpu.async_copy(kft_hbm.at[idx_v.at[k]], rows_v.at[k],
                             sem).wait()
            pltpu.sync_copy(rows_v.at[k], rp.at[pl.ds(row0, _CP)])
        return carry

    lax.fori_loop(0, _PW // _CP, chunk_body, 0)


def _sc_gather(kft2, i1, i2, i3):
    mesh = plsc.VectorSubcoreMesh(core_axis_name="c", subcore_axis_name="s")
    rows_t = jax.ShapeDtypeStruct((_B * _N, _C2), jnp.float32)
    f = functools.partial(
        pl.kernel,
        out_type=[rows_t, rows_t, rows_t],
        mesh=mesh,
        scratch_types=[
            pltpu.VMEM((3, _CP), jnp.int32),
            pltpu.VMEM((3, _CP, _C2), jnp.float32),
            pltpu.SemaphoreType.DMA,
        ],
    )(_sc_gather_body)
    return f(kft2, i1, i2, i3)


def _mlp_body(r1_ref, r2_ref, r3_ref, w_ref, uf_ref, w1m_ref, w2m_ref,
              out_ref):
    w = w_ref[0]
    interp = (w[:, 0:1] * r1_ref[...]
              + w[:, 1:2] * r2_ref[...]
              + w[:, 2:3] * r3_ref[...])
    w1a = w1m_ref[:_C2, :]
    w1b = w1m_ref[_C2:, :]
    h = lax.dot_general(w1a, interp, (((0,), (1,)), ((), ())),
                        preferred_element_type=jnp.float32)
    h = h + lax.dot_general(w1b, uf_ref[0], (((0,), (0,)), ((), ())),
                            preferred_element_type=jnp.float32)
    h = jnp.maximum(h, 0.0)
    h = lax.dot_general(w2m_ref[...], h, (((0,), (0,)), ((), ())),
                        preferred_element_type=jnp.float32)
    out_ref[0] = jnp.maximum(h, 0.0)


def _mlp(r1, r2, r3, wts, unknow_feats, W1, W2):
    nb = _N // _NBLK
    return pl.pallas_call(
        _mlp_body,
        grid=(_B, nb),
        in_specs=[
            pl.BlockSpec((_NBLK, _C2), lambda b, n: (b * nb + n, 0)),
            pl.BlockSpec((_NBLK, _C2), lambda b, n: (b * nb + n, 0)),
            pl.BlockSpec((_NBLK, _C2), lambda b, n: (b * nb + n, 0)),
            pl.BlockSpec((1, _NBLK, 3), lambda b, n: (b, n, 0)),
            pl.BlockSpec((1, _C1, _NBLK), lambda b, n: (b, 0, n)),
            pl.BlockSpec((_C1 + _C2, 256), lambda b, n: (0, 0)),
            pl.BlockSpec((256, 256), lambda b, n: (0, 0)),
        ],
        out_specs=pl.BlockSpec((1, 256, _NBLK), lambda b, n: (b, 0, n)),
        out_shape=jax.ShapeDtypeStruct((_B, 256, _N), jnp.float32),
    )(r1, r2, r3, wts, unknow_feats, W1, W2)


@jax.jit
def kernel(unknown, known, unknow_feats, known_feats, W1, W2):
    kt = known.swapaxes(1, 2)
    kft2 = known_feats.swapaxes(1, 2).reshape(_B * _M, _C2)

    i1, i2, i3, wts = _three_nn(unknown, kt)
    r1, r2, r3 = _sc_gather(kft2, i1, i2, i3)
    return _mlp(r1, r2, r3, wts, unknow_feats, W1, W2)


# speedup vs baseline: 1.4530x; 1.1407x over previous
import functools
import jax
import jax.numpy as jnp
from jax import lax
from jax.experimental import pallas as pl
from jax.experimental.pallas import tpu as pltpu
from jax.experimental.pallas import tpu_sc as plsc

_B, _N, _M, _C1, _C2 = 8, 4096, 1024, 128, 256
_NBLK = 512
_BIG = 3.0e38

_G = 2
_BG = _B // _G
_NW = 32
_PW = _BG * _N // _NW
_CP = 128
_WPB = _N // _PW


def _nn_body(u_ref, kt_ref, i1_ref, i2_ref, i3_ref, w_ref, *, b0):
    b = pl.program_id(0) + b0
    u = u_ref[0]
    kt = kt_ref[0]

    d2 = jnp.zeros((_NBLK, _M), jnp.float32)
    for d in range(3):
        diff = u[:, d:d + 1] - kt[d:d + 1, :]
        d2 = d2 + diff * diff

    ids = lax.broadcasted_iota(jnp.int32, (_NBLK, _M), 1)
    cur = d2
    mins = []
    idxs = []
    for _ in range(3):
        m = jnp.min(cur, axis=1, keepdims=True)
        i = jnp.min(jnp.where(cur == m, ids, _M), axis=1, keepdims=True)
        mins.append(m)
        idxs.append(i)
        cur = jnp.where(ids == i, _BIG, cur)

    r1 = 1.0 / (mins[0] + 1e-8)
    r2 = 1.0 / (mins[1] + 1e-8)
    r3 = 1.0 / (mins[2] + 1e-8)
    norm = r1 + r2 + r3

    i1_ref[0, 0] = jnp.reshape(idxs[0] + b * _M, (_NBLK,))
    i2_ref[0, 0] = jnp.reshape(idxs[1] + b * _M, (_NBLK,))
    i3_ref[0, 0] = jnp.reshape(idxs[2] + b * _M, (_NBLK,))
    w_ref[0] = jnp.concatenate([r1 / norm, r2 / norm, r3 / norm], axis=1)


def _three_nn(unknown, kt, b0):
    return pl.pallas_call(
        functools.partial(_nn_body, b0=b0),
        grid=(_BG, _N // _NBLK),
        in_specs=[
            pl.BlockSpec((1, _NBLK, 3), lambda b, n: (b + b0, n, 0)),
            pl.BlockSpec((1, 3, _M), lambda b, n: (b + b0, 0, 0)),
        ],
        out_specs=[
            pl.BlockSpec((1, 1, _NBLK), lambda b, n: (b, 0, n)),
            pl.BlockSpec((1, 1, _NBLK), lambda b, n: (b, 0, n)),
            pl.BlockSpec((1, 1, _NBLK), lambda b, n: (b, 0, n)),
            pl.BlockSpec((1, _NBLK, 3), lambda b, n: (b, n, 0)),
        ],
        out_shape=[
            jax.ShapeDtypeStruct((_BG, 1, _N), jnp.int32),
            jax.ShapeDtypeStruct((_BG, 1, _N), jnp.int32),
            jax.ShapeDtypeStruct((_BG, 1, _N), jnp.int32),
            jax.ShapeDtypeStruct((_BG, _N, 3), jnp.float32),
        ],
    )(unknown, kt)


def _sc_gather_body(kft_hbm, i1_hbm, i2_hbm, i3_hbm,
                    r1_hbm, r2_hbm, r3_hbm,
                    idx_v, rows_v, sem):
    wid = lax.axis_index("s") * 2 + lax.axis_index("c")
    bi = wid // _WPB
    col0 = (wid % _WPB) * _PW

    def chunk_body(ci, carry):
        col = col0 + ci * _CP
        row0 = bi * _N + col
        for k, (ip, rp) in enumerate(((i1_hbm, r1_hbm), (i2_hbm, r2_hbm),
                                      (i3_hbm, r3_hbm))):
            pltpu.sync_copy(ip.at[bi, 0, pl.ds(col, _CP)], idx_v.at[k])
            pltpu.async_copy(kft_hbm.at[idx_v.at[k]], rows_v.at[k],
                             sem).wait()
            pltpu.sync_copy(rows_v.at[k], rp.at[pl.ds(row0, _CP)])
        return carry

    lax.fori_loop(0, _PW // _CP, chunk_body, 0)


def _sc_gather(kft2, i1, i2, i3):
    mesh = plsc.VectorSubcoreMesh(core_axis_name="c", subcore_axis_name="s")
    rows_t = jax.ShapeDtypeStruct((_BG * _N, _C2), jnp.float32)
    f = functools.partial(
        pl.kernel,
        out_type=[rows_t, rows_t, rows_t],
        mesh=mesh,
        scratch_types=[
            pltpu.VMEM((3, _CP), jnp.int32),
            pltpu.VMEM((3, _CP, _C2), jnp.float32),
            pltpu.SemaphoreType.DMA,
        ],
    )(_sc_gather_body)
    return f(kft2, i1, i2, i3)


def _mlp_body(r1_ref, r2_ref, r3_ref, w_ref, uf_ref, w1m_ref, w2m_ref,
              out_ref):
    w = w_ref[0]
    interp = (w[:, 0:1] * r1_ref[...]
              + w[:, 1:2] * r2_ref[...]
              + w[:, 2:3] * r3_ref[...])
    w1a = w1m_ref[:_C2, :]
    w1b = w1m_ref[_C2:, :]
    h = lax.dot_general(w1a, interp, (((0,), (1,)), ((), ())),
                        preferred_element_type=jnp.float32)
    h = h + lax.dot_general(w1b, uf_ref[0], (((0,), (0,)), ((), ())),
                            preferred_element_type=jnp.float32)
    h = jnp.maximum(h, 0.0)
    h = lax.dot_general(w2m_ref[...], h, (((0,), (0,)), ((), ())),
                        preferred_element_type=jnp.float32)
    out_ref[0] = jnp.maximum(h, 0.0)


def _mlp(r1, r2, r3, wts, unknow_feats, W1, W2, b0):
    nb = _N // _NBLK
    return pl.pallas_call(
        _mlp_body,
        grid=(_BG, nb),
        in_specs=[
            pl.BlockSpec((_NBLK, _C2), lambda b, n: (b * nb + n, 0)),
            pl.BlockSpec((_NBLK, _C2), lambda b, n: (b * nb + n, 0)),
            pl.BlockSpec((_NBLK, _C2), lambda b, n: (b * nb + n, 0)),
            pl.BlockSpec((1, _NBLK, 3), lambda b, n: (b, n, 0)),
            pl.BlockSpec((1, _C1, _NBLK), lambda b, n: (b + b0, 0, n)),
            pl.BlockSpec((_C1 + _C2, 256), lambda b, n: (0, 0)),
            pl.BlockSpec((256, 256), lambda b, n: (0, 0)),
        ],
        out_specs=pl.BlockSpec((1, 256, _NBLK), lambda b, n: (b, 0, n)),
        out_shape=jax.ShapeDtypeStruct((_BG, 256, _N), jnp.float32),
    )(r1, r2, r3, wts, unknow_feats, W1, W2)


@jax.jit
def kernel(unknown, known, unknow_feats, known_feats, W1, W2):
    kt = known.swapaxes(1, 2)
    kft2 = known_feats.swapaxes(1, 2).reshape(_B * _M, _C2)

    nn = [_three_nn(unknown, kt, g * _BG) for g in range(_G)]
    rows = [_sc_gather(kft2, i1, i2, i3) for (i1, i2, i3, _) in nn]
    outs = [
        _mlp(r1, r2, r3, nn[g][3], unknow_feats, W1, W2, g * _BG)
        for g, (r1, r2, r3) in enumerate(rows)
    ]
    return jnp.concatenate(outs, axis=0)


# speedup vs baseline: 1.5532x; 1.0689x over previous
import functools
import jax
import jax.numpy as jnp
from jax import lax
from jax.experimental import pallas as pl
from jax.experimental.pallas import tpu as pltpu
from jax.experimental.pallas import tpu_sc as plsc

_B, _N, _M, _C1, _C2 = 8, 4096, 1024, 128, 256
_NBLK = 512
_BIG = 3.0e38

_G = 2
_BG = _B // _G
_NW = 32
_PW = _BG * _N // _NW
_CP = 128
_WPB = _N // _PW


def _nn_body(u_ref, kt_ref, ip_ref, w_ref, *, b0):
    b = pl.program_id(0) + b0
    u = u_ref[0]
    kt = kt_ref[0]

    d2 = jnp.zeros((_NBLK, _M), jnp.float32)
    for d in range(3):
        diff = u[:, d:d + 1] - kt[d:d + 1, :]
        d2 = d2 + diff * diff

    ids = lax.broadcasted_iota(jnp.int32, (_NBLK, _M), 1)
    cur = d2
    mins = []
    idxs = []
    for _ in range(3):
        m = jnp.min(cur, axis=1, keepdims=True)
        eq = cur == m
        i = jnp.min(jnp.where(eq, ids, _M), axis=1, keepdims=True)
        mins.append(m)
        idxs.append(i)
        cur = jnp.where(eq, _BIG, cur)

    r1 = 1.0 / (mins[0] + 1e-8)
    r2 = 1.0 / (mins[1] + 1e-8)
    r3 = 1.0 / (mins[2] + 1e-8)
    norm = r1 + r2 + r3

    packed = (idxs[0] << 20) + (idxs[1] << 10) + idxs[2]
    ip_ref[0, 0] = jnp.reshape(packed, (_NBLK,))
    w_ref[0] = jnp.concatenate([r1 / norm, r2 / norm, r3 / norm], axis=1)


def _three_nn(unknown, kt, b0):
    return pl.pallas_call(
        functools.partial(_nn_body, b0=b0),
        grid=(_BG, _N // _NBLK),
        in_specs=[
            pl.BlockSpec((1, _NBLK, 3), lambda b, n: (b + b0, n, 0)),
            pl.BlockSpec((1, 3, _M), lambda b, n: (b + b0, 0, 0)),
        ],
        out_specs=[
            pl.BlockSpec((1, 1, _NBLK), lambda b, n: (b, 0, n)),
            pl.BlockSpec((1, _NBLK, 3), lambda b, n: (b, n, 0)),
        ],
        out_shape=[
            jax.ShapeDtypeStruct((_BG, 1, _N), jnp.int32),
            jax.ShapeDtypeStruct((_BG, _N, 3), jnp.float32),
        ],
    )(unknown, kt)


def _sc_gather_body(kft_hbm, ip_hbm, r1_hbm, r2_hbm, r3_hbm,
                    pk_v, idx_v, rows_v, sem, *, b0):
    wid = lax.axis_index("s") * 2 + lax.axis_index("c")
    bi = wid // _WPB
    col0 = (wid % _WPB) * _PW

    def chunk_body(ci, carry):
        col = col0 + ci * _CP
        row0 = bi * _N + col
        pltpu.sync_copy(ip_hbm.at[bi, 0, pl.ds(col, _CP)], pk_v)
        base = (b0 + bi) * _M
        for g in range(_CP // 16):
            sl = pl.ds(16 * g, 16)
            pk = pk_v[sl]
            idx_v[0, sl] = (pk >> 20) + base
            idx_v[1, sl] = ((pk >> 10) & 1023) + base
            idx_v[2, sl] = (pk & 1023) + base
        for k, rp in enumerate((r1_hbm, r2_hbm, r3_hbm)):
            pltpu.async_copy(kft_hbm.at[idx_v.at[k]], rows_v.at[k],
                             sem).wait()
            pltpu.sync_copy(rows_v.at[k], rp.at[pl.ds(row0, _CP)])
        return carry

    lax.fori_loop(0, _PW // _CP, chunk_body, 0)


def _sc_gather(kft2, ip, b0):
    mesh = plsc.VectorSubcoreMesh(core_axis_name="c", subcore_axis_name="s")
    rows_t = jax.ShapeDtypeStruct((_BG * _N, _C2), jnp.float32)
    f = functools.partial(
        pl.kernel,
        out_type=[rows_t, rows_t, rows_t],
        mesh=mesh,
        scratch_types=[
            pltpu.VMEM((_CP,), jnp.int32),
            pltpu.VMEM((3, _CP), jnp.int32),
            pltpu.VMEM((3, _CP, _C2), jnp.float32),
            pltpu.SemaphoreType.DMA,
        ],
    )(functools.partial(_sc_gather_body, b0=b0))
    return f(kft2, ip)


def _mlp_body(r1_ref, r2_ref, r3_ref, w_ref, uf_ref, w1m_ref, w2m_ref,
              out_ref):
    w = w_ref[0]
    interp = (w[:, 0:1] * r1_ref[...]
              + w[:, 1:2] * r2_ref[...]
              + w[:, 2:3] * r3_ref[...])
    w1a = w1m_ref[:_C2, :]
    w1b = w1m_ref[_C2:, :]
    h = lax.dot_general(w1a, interp, (((0,), (1,)), ((), ())),
                        preferred_element_type=jnp.float32)
    h = h + lax.dot_general(w1b, uf_ref[0], (((0,), (0,)), ((), ())),
                            preferred_element_type=jnp.float32)
    h = jnp.maximum(h, 0.0)
    h = lax.dot_general(w2m_ref[...], h, (((0,), (0,)), ((), ())),
                        preferred_element_type=jnp.float32)
    out_ref[0] = jnp.maximum(h, 0.0)


def _mlp(r1, r2, r3, wts, unknow_feats, W1, W2, b0):
    nb = _N // _NBLK
    return pl.pallas_call(
        _mlp_body,
        grid=(_BG, nb),
        in_specs=[
            pl.BlockSpec((_NBLK, _C2), lambda b, n: (b * nb + n, 0)),
            pl.BlockSpec((_NBLK, _C2), lambda b, n: (b * nb + n, 0)),
            pl.BlockSpec((_NBLK, _C2), lambda b, n: (b * nb + n, 0)),
            pl.BlockSpec((1, _NBLK, 3), lambda b, n: (b, n, 0)),
            pl.BlockSpec((1, _C1, _NBLK), lambda b, n: (b + b0, 0, n)),
            pl.BlockSpec((_C1 + _C2, 256), lambda b, n: (0, 0)),
            pl.BlockSpec((256, 256), lambda b, n: (0, 0)),
        ],
        out_specs=pl.BlockSpec((1, 256, _NBLK), lambda b, n: (b, 0, n)),
        out_shape=jax.ShapeDtypeStruct((_BG, 256, _N), jnp.float32),
    )(r1, r2, r3, wts, unknow_feats, W1, W2)


@jax.jit
def kernel(unknown, known, unknow_feats, known_feats, W1, W2):
    kt = known.swapaxes(1, 2)
    kft2 = known_feats.swapaxes(1, 2).reshape(_B * _M, _C2)

    nn = [_three_nn(unknown, kt, g * _BG) for g in range(_G)]
    rows = [_sc_gather(kft2, ip, g * _BG) for g, (ip, _) in enumerate(nn)]
    outs = [
        _mlp(r1, r2, r3, nn[g][1], unknow_feats, W1, W2, g * _BG)
        for g, (r1, r2, r3) in enumerate(rows)
    ]
    return jnp.concatenate(outs, axis=0)


# speedup vs baseline: 1.6595x; 1.0684x over previous
import functools
import jax
import jax.numpy as jnp
from jax import lax
from jax.experimental import pallas as pl
from jax.experimental.pallas import tpu as pltpu
from jax.experimental.pallas import tpu_sc as plsc

_B, _N, _M, _C1, _C2 = 8, 4096, 1024, 128, 256
_NBLK = 512
_BIG = 3.0e38

_G = 2
_BG = _B // _G
_NW = 32
_PW = _BG * _N // _NW
_CP = 128
_WPB = _N // _PW
_MT = _M // (_N // _NBLK)


def _nn_body(u_ref, kt_ref, kf_ref, ip_ref, w_ref, kft2_ref, *, b0):
    b = pl.program_id(0) + b0
    u = u_ref[0]
    kt = kt_ref[0]

    d2 = jnp.zeros((_NBLK, _M), jnp.float32)
    for d in range(3):
        diff = u[:, d:d + 1] - kt[d:d + 1, :]
        d2 = d2 + diff * diff

    ids = lax.broadcasted_iota(jnp.int32, (_NBLK, _M), 1)
    cur = d2
    mins = []
    idxs = []
    for _ in range(3):
        m = jnp.min(cur, axis=1, keepdims=True)
        eq = cur == m
        i = jnp.min(jnp.where(eq, ids, _M), axis=1, keepdims=True)
        mins.append(m)
        idxs.append(i)
        cur = jnp.where(eq, _BIG, cur)

    r1 = 1.0 / (mins[0] + 1e-8)
    r2 = 1.0 / (mins[1] + 1e-8)
    r3 = 1.0 / (mins[2] + 1e-8)
    norm = r1 + r2 + r3

    packed = (idxs[0] << 20) + (idxs[1] << 10) + idxs[2]
    ip_ref[0, 0] = jnp.reshape(packed, (_NBLK,))
    w_ref[0] = jnp.concatenate([r1 / norm, r2 / norm, r3 / norm], axis=1)
    kft2_ref[...] = kf_ref[0].T


def _three_nn(unknown, kt, known_feats, b0):
    return pl.pallas_call(
        functools.partial(_nn_body, b0=b0),
        grid=(_BG, _N // _NBLK),
        in_specs=[
            pl.BlockSpec((1, _NBLK, 3), lambda b, n: (b + b0, n, 0)),
            pl.BlockSpec((1, 3, _M), lambda b, n: (b + b0, 0, 0)),
            pl.BlockSpec((1, _C2, _MT), lambda b, n: (b + b0, 0, n)),
        ],
        out_specs=[
            pl.BlockSpec((1, 1, _NBLK), lambda b, n: (b, 0, n)),
            pl.BlockSpec((1, _NBLK, 3), lambda b, n: (b, n, 0)),
            pl.BlockSpec((_MT, _C2), lambda b, n: (b * (_N // _NBLK) + n, 0)),
        ],
        out_shape=[
            jax.ShapeDtypeStruct((_BG, 1, _N), jnp.int32),
            jax.ShapeDtypeStruct((_BG, _N, 3), jnp.float32),
            jax.ShapeDtypeStruct((_BG * _M, _C2), jnp.float32),
        ],
    )(unknown, kt, known_feats)


def _sc_gather_body(kft_hbm, ip_hbm, r1_hbm, r2_hbm, r3_hbm,
                    pk_v, idx_v, rows_v, sem):
    wid = lax.axis_index("s") * 2 + lax.axis_index("c")
    bi = wid // _WPB
    col0 = (wid % _WPB) * _PW

    def chunk_body(ci, carry):
        col = col0 + ci * _CP
        row0 = bi * _N + col
        pltpu.sync_copy(ip_hbm.at[bi, 0, pl.ds(col, _CP)], pk_v)
        base = bi * _M
        for g in range(_CP // 16):
            sl = pl.ds(16 * g, 16)
            pk = pk_v[sl]
            idx_v[0, sl] = (pk >> 20) + base
            idx_v[1, sl] = ((pk >> 10) & 1023) + base
            idx_v[2, sl] = (pk & 1023) + base
        for k, rp in enumerate((r1_hbm, r2_hbm, r3_hbm)):
            pltpu.async_copy(kft_hbm.at[idx_v.at[k]], rows_v.at[k],
                             sem).wait()
            pltpu.sync_copy(rows_v.at[k], rp.at[pl.ds(row0, _CP)])
        return carry

    lax.fori_loop(0, _PW // _CP, chunk_body, 0)


def _sc_gather(kft2, ip):
    mesh = plsc.VectorSubcoreMesh(core_axis_name="c", subcore_axis_name="s")
    rows_t = jax.ShapeDtypeStruct((_BG * _N, _C2), jnp.float32)
    f = functools.partial(
        pl.kernel,
        out_type=[rows_t, rows_t, rows_t],
        mesh=mesh,
        scratch_types=[
            pltpu.VMEM((_CP,), jnp.int32),
            pltpu.VMEM((3, _CP), jnp.int32),
            pltpu.VMEM((3, _CP, _C2), jnp.float32),
            pltpu.SemaphoreType.DMA,
        ],
    )(_sc_gather_body)
    return f(kft2, ip)


def _mlp_body(r1_ref, r2_ref, r3_ref, w_ref, uf_ref, w1m_ref, w2m_ref,
              *refs):
    out_ref = refs[-1]
    w = w_ref[0]
    interp = (w[:, 0:1] * r1_ref[...]
              + w[:, 1:2] * r2_ref[...]
              + w[:, 2:3] * r3_ref[...])
    w1a = w1m_ref[:_C2, :]
    w1b = w1m_ref[_C2:, :]
    h = lax.dot_general(w1a, interp, (((0,), (1,)), ((), ())),
                        preferred_element_type=jnp.float32)
    h = h + lax.dot_general(w1b, uf_ref[0], (((0,), (0,)), ((), ())),
                            preferred_element_type=jnp.float32)
    h = jnp.maximum(h, 0.0)
    h = lax.dot_general(w2m_ref[...], h, (((0,), (0,)), ((), ())),
                        preferred_element_type=jnp.float32)
    out_ref[0] = jnp.maximum(h, 0.0)


def _mlp(r1, r2, r3, wts, unknow_feats, W1, W2, prev, b0):
    nb = _N // _NBLK
    in_specs = [
        pl.BlockSpec((_NBLK, _C2), lambda b, n: (b * nb + n, 0)),
        pl.BlockSpec((_NBLK, _C2), lambda b, n: (b * nb + n, 0)),
        pl.BlockSpec((_NBLK, _C2), lambda b, n: (b * nb + n, 0)),
        pl.BlockSpec((1, _NBLK, 3), lambda b, n: (b, n, 0)),
        pl.BlockSpec((1, _C1, _NBLK), lambda b, n: (b + b0, 0, n)),
        pl.BlockSpec((_C1 + _C2, 256), lambda b, n: (0, 0)),
        pl.BlockSpec((256, 256), lambda b, n: (0, 0)),
    ]
    args = [r1, r2, r3, wts, unknow_feats, W1, W2]
    aliases = {}
    if prev is not None:
        in_specs.append(pl.BlockSpec(memory_space=pl.ANY))
        args.append(prev)
        aliases = {7: 0}
    return pl.pallas_call(
        _mlp_body,
        grid=(_BG, nb),
        in_specs=in_specs,
        out_specs=pl.BlockSpec((1, 256, _NBLK), lambda b, n: (b + b0, 0, n)),
        out_shape=jax.ShapeDtypeStruct((_B, 256, _N), jnp.float32),
        input_output_aliases=aliases,
    )(*args)


@jax.jit
def kernel(unknown, known, unknow_feats, known_feats, W1, W2):
    kt = known.swapaxes(1, 2)

    nn = [_three_nn(unknown, kt, known_feats, g * _BG) for g in range(_G)]
    rows = [_sc_gather(kft2, ip) for (ip, _, kft2) in nn]
    out = None
    for g, (r1, r2, r3) in enumerate(rows):
        out = _mlp(r1, r2, r3, nn[g][1], unknow_feats, W1, W2, out,
                   g * _BG)
    return out


# speedup vs baseline: 1.8029x; 1.0865x over previous
import functools
import jax
import jax.numpy as jnp
from jax import lax
from jax.experimental import pallas as pl
from jax.experimental.pallas import tpu as pltpu
from jax.experimental.pallas import tpu_sc as plsc

_B, _N, _M, _C1, _C2 = 8, 4096, 1024, 128, 256
_NBLK = 512
_BIG = 3.0e38

_G = 2
_BG = _B // _G
_NW = 32
_PW = _BG * _N // _NW
_CP = 128
_WPB = _N // _PW
_MT = _M // (_N // _NBLK)


def _nn_body(u_ref, kt_ref, kf_ref, ip_ref, w_ref, kft2_ref, *, b0):
    b = pl.program_id(0) + b0
    u = u_ref[0]
    kt = kt_ref[0]

    d2 = jnp.zeros((_NBLK, _M), jnp.float32)
    for d in range(3):
        diff = u[:, d:d + 1] - kt[d:d + 1, :]
        d2 = d2 + diff * diff

    ids = lax.broadcasted_iota(jnp.int32, (_NBLK, _M), 1).astype(jnp.float32)
    cur = d2
    mins = []
    idxs = []
    for _ in range(3):
        m = jnp.min(cur, axis=1, keepdims=True)
        eq = cur == m
        i = jnp.min(jnp.where(eq, ids, jnp.float32(_M)), axis=1,
                    keepdims=True)
        mins.append(m)
        idxs.append(i.astype(jnp.int32))
        cur = jnp.where(eq, _BIG, cur)

    r1 = 1.0 / (mins[0] + 1e-8)
    r2 = 1.0 / (mins[1] + 1e-8)
    r3 = 1.0 / (mins[2] + 1e-8)
    norm = r1 + r2 + r3

    packed = (idxs[0] << 20) + (idxs[1] << 10) + idxs[2]
    ip_ref[0, 0] = jnp.reshape(packed, (_NBLK,))
    w_ref[0] = jnp.concatenate([r1 / norm, r2 / norm, r3 / norm], axis=1)
    kft2_ref[...] = kf_ref[0].T


def _three_nn(unknown, kt, known_feats, b0):
    return pl.pallas_call(
        functools.partial(_nn_body, b0=b0),
        grid=(_BG, _N // _NBLK),
        in_specs=[
            pl.BlockSpec((1, _NBLK, 3), lambda b, n: (b + b0, n, 0)),
            pl.BlockSpec((1, 3, _M), lambda b, n: (b + b0, 0, 0)),
            pl.BlockSpec((1, _C2, _MT), lambda b, n: (b + b0, 0, n)),
        ],
        out_specs=[
            pl.BlockSpec((1, 1, _NBLK), lambda b, n: (b, 0, n)),
            pl.BlockSpec((1, _NBLK, 3), lambda b, n: (b, n, 0)),
            pl.BlockSpec((_MT, _C2), lambda b, n: (b * (_N // _NBLK) + n, 0)),
        ],
        out_shape=[
            jax.ShapeDtypeStruct((_BG, 1, _N), jnp.int32),
            jax.ShapeDtypeStruct((_BG, _N, 3), jnp.float32),
            jax.ShapeDtypeStruct((_BG * _M, _C2), jnp.float32),
        ],
    )(unknown, kt, known_feats)


def _sc_gather_body(kft_hbm, ip_hbm, r1_hbm, r2_hbm, r3_hbm,
                    pk_v, idx_v, rows_v, sem):
    wid = lax.axis_index("s") * 2 + lax.axis_index("c")
    bi = wid // _WPB
    col0 = (wid % _WPB) * _PW

    def chunk_body(ci, carry):
        col = col0 + ci * _CP
        row0 = bi * _N + col
        pltpu.sync_copy(ip_hbm.at[bi, 0, pl.ds(col, _CP)], pk_v)
        base = bi * _M
        for g in range(_CP // 16):
            sl = pl.ds(16 * g, 16)
            pk = pk_v[sl]
            idx_v[0, sl] = (pk >> 20) + base
            idx_v[1, sl] = ((pk >> 10) & 1023) + base
            idx_v[2, sl] = (pk & 1023) + base
        for k, rp in enumerate((r1_hbm, r2_hbm, r3_hbm)):
            pltpu.async_copy(kft_hbm.at[idx_v.at[k]], rows_v.at[k],
                             sem).wait()
            pltpu.sync_copy(rows_v.at[k], rp.at[pl.ds(row0, _CP)])
        return carry

    lax.fori_loop(0, _PW // _CP, chunk_body, 0)


def _sc_gather(kft2, ip):
    mesh = plsc.VectorSubcoreMesh(core_axis_name="c", subcore_axis_name="s")
    rows_t = jax.ShapeDtypeStruct((_BG * _N, _C2), jnp.float32)
    f = functools.partial(
        pl.kernel,
        out_type=[rows_t, rows_t, rows_t],
        mesh=mesh,
        scratch_types=[
            pltpu.VMEM((_CP,), jnp.int32),
            pltpu.VMEM((3, _CP), jnp.int32),
            pltpu.VMEM((3, _CP, _C2), jnp.float32),
            pltpu.SemaphoreType.DMA,
        ],
    )(_sc_gather_body)
    return f(kft2, ip)


def _mlp_body(r1_ref, r2_ref, r3_ref, w_ref, uf_ref, w1m_ref, w2m_ref,
              *refs):
    out_ref = refs[-1]
    w = w_ref[0]
    interp = (w[:, 0:1] * r1_ref[...]
              + w[:, 1:2] * r2_ref[...]
              + w[:, 2:3] * r3_ref[...])
    w1a = w1m_ref[:_C2, :]
    w1b = w1m_ref[_C2:, :]
    h = lax.dot_general(w1a, interp, (((0,), (1,)), ((), ())),
                        preferred_element_type=jnp.float32)
    h = h + lax.dot_general(w1b, uf_ref[0], (((0,), (0,)), ((), ())),
                            preferred_element_type=jnp.float32)
    h = jnp.maximum(h, 0.0)
    h = lax.dot_general(w2m_ref[...], h, (((0,), (0,)), ((), ())),
                        preferred_element_type=jnp.float32)
    out_ref[0] = jnp.maximum(h, 0.0)


def _mlp(r1, r2, r3, wts, unknow_feats, W1, W2, prev, b0):
    nb = _N // _NBLK
    in_specs = [
        pl.BlockSpec((_NBLK, _C2), lambda b, n: (b * nb + n, 0)),
        pl.BlockSpec((_NBLK, _C2), lambda b, n: (b * nb + n, 0)),
        pl.BlockSpec((_NBLK, _C2), lambda b, n: (b * nb + n, 0)),
        pl.BlockSpec((1, _NBLK, 3), lambda b, n: (b, n, 0)),
        pl.BlockSpec((1, _C1, _NBLK), lambda b, n: (b + b0, 0, n)),
        pl.BlockSpec((_C1 + _C2, 256), lambda b, n: (0, 0)),
        pl.BlockSpec((256, 256), lambda b, n: (0, 0)),
    ]
    args = [r1, r2, r3, wts, unknow_feats, W1, W2]
    aliases = {}
    if prev is not None:
        in_specs.append(pl.BlockSpec(memory_space=pl.ANY))
        args.append(prev)
        aliases = {7: 0}
    return pl.pallas_call(
        _mlp_body,
        grid=(_BG, nb),
        in_specs=in_specs,
        out_specs=pl.BlockSpec((1, 256, _NBLK), lambda b, n: (b + b0, 0, n)),
        out_shape=jax.ShapeDtypeStruct((_B, 256, _N), jnp.float32),
        input_output_aliases=aliases,
    )(*args)


@jax.jit
def kernel(unknown, known, unknow_feats, known_feats, W1, W2):
    kt = known.swapaxes(1, 2)

    nn = [_three_nn(unknown, kt, known_feats, g * _BG) for g in range(_G)]
    rows = [_sc_gather(kft2, ip) for (ip, _, kft2) in nn]
    out = None
    for g, (r1, r2, r3) in enumerate(rows):
        out = _mlp(r1, r2, r3, nn[g][1], unknow_feats, W1, W2, out,
                   g * _BG)
    return out


# speedup vs baseline: 1.9380x; 1.0749x over previous
import functools
import jax
import jax.numpy as jnp
from jax import lax
from jax.experimental import pallas as pl
from jax.experimental.pallas import tpu as pltpu
from jax.experimental.pallas import tpu_sc as plsc

_B, _N, _M, _C1, _C2 = 8, 4096, 1024, 128, 256
_NBLK = 1024
_BIG = 3.0e38

_G = 2
_BG = _B // _G
_NW = 32
_PW = _BG * _N // _NW
_CP = 128
_WPB = _N // _PW
_MT = _M // (_N // _NBLK)


def _nn_body(u_ref, kt_ref, kf_ref, ip_ref, w_ref, kft2_ref, *, b0):
    b = pl.program_id(0) + b0
    u = u_ref[0]
    kt = kt_ref[0]

    d2 = jnp.zeros((_NBLK, _M), jnp.float32)
    for d in range(3):
        diff = u[:, d:d + 1] - kt[d:d + 1, :]
        d2 = d2 + diff * diff

    ids = lax.broadcasted_iota(jnp.int32, (_NBLK, _M), 1).astype(jnp.float32)
    cur = d2
    mins = []
    idxs = []
    for _ in range(3):
        m = jnp.min(cur, axis=1, keepdims=True)
        eq = cur == m
        i = jnp.min(jnp.where(eq, ids, jnp.float32(_M)), axis=1,
                    keepdims=True)
        mins.append(m)
        idxs.append(i.astype(jnp.int32))
        cur = jnp.where(eq, _BIG, cur)

    r1 = 1.0 / (mins[0] + 1e-8)
    r2 = 1.0 / (mins[1] + 1e-8)
    r3 = 1.0 / (mins[2] + 1e-8)
    norm = r1 + r2 + r3

    packed = (idxs[0] << 20) + (idxs[1] << 10) + idxs[2]
    ip_ref[0, 0] = jnp.reshape(packed, (_NBLK,))
    w_ref[0] = jnp.concatenate([r1 / norm, r2 / norm, r3 / norm], axis=1)
    kft2_ref[...] = kf_ref[0].T


def _three_nn(unknown, kt, known_feats, b0):
    return pl.pallas_call(
        functools.partial(_nn_body, b0=b0),
        grid=(_BG, _N // _NBLK),
        in_specs=[
            pl.BlockSpec((1, _NBLK, 3), lambda b, n: (b + b0, n, 0)),
            pl.BlockSpec((1, 3, _M), lambda b, n: (b + b0, 0, 0)),
            pl.BlockSpec((1, _C2, _MT), lambda b, n: (b + b0, 0, n)),
        ],
        out_specs=[
            pl.BlockSpec((1, 1, _NBLK), lambda b, n: (b, 0, n)),
            pl.BlockSpec((1, _NBLK, 3), lambda b, n: (b, n, 0)),
            pl.BlockSpec((_MT, _C2), lambda b, n: (b * (_N // _NBLK) + n, 0)),
        ],
        out_shape=[
            jax.ShapeDtypeStruct((_BG, 1, _N), jnp.int32),
            jax.ShapeDtypeStruct((_BG, _N, 3), jnp.float32),
            jax.ShapeDtypeStruct((_BG * _M, _C2), jnp.float32),
        ],
    )(unknown, kt, known_feats)


def _sc_gather_body(kft_hbm, ip_hbm, r1_hbm, r2_hbm, r3_hbm,
                    pk_v, idx_v, rows_v, sem):
    wid = lax.axis_index("s") * 2 + lax.axis_index("c")
    bi = wid // _WPB
    col0 = (wid % _WPB) * _PW

    def chunk_body(ci, carry):
        col = col0 + ci * _CP
        row0 = bi * _N + col
        pltpu.sync_copy(ip_hbm.at[bi, 0, pl.ds(col, _CP)], pk_v)
        base = bi * _M
        for g in range(_CP // 16):
            sl = pl.ds(16 * g, 16)
            pk = pk_v[sl]
            idx_v[0, sl] = (pk >> 20) + base
            idx_v[1, sl] = ((pk >> 10) & 1023) + base
            idx_v[2, sl] = (pk & 1023) + base
        for k, rp in enumerate((r1_hbm, r2_hbm, r3_hbm)):
            pltpu.async_copy(kft_hbm.at[idx_v.at[k]], rows_v.at[k],
                             sem).wait()
            pltpu.sync_copy(rows_v.at[k], rp.at[pl.ds(row0, _CP)])
        return carry

    lax.fori_loop(0, _PW // _CP, chunk_body, 0)


def _sc_gather(kft2, ip):
    mesh = plsc.VectorSubcoreMesh(core_axis_name="c", subcore_axis_name="s")
    rows_t = jax.ShapeDtypeStruct((_BG * _N, _C2), jnp.float32)
    f = functools.partial(
        pl.kernel,
        out_type=[rows_t, rows_t, rows_t],
        mesh=mesh,
        scratch_types=[
            pltpu.VMEM((_CP,), jnp.int32),
            pltpu.VMEM((3, _CP), jnp.int32),
            pltpu.VMEM((3, _CP, _C2), jnp.float32),
            pltpu.SemaphoreType.DMA,
        ],
    )(_sc_gather_body)
    return f(kft2, ip)


def _mlp_body(r1_ref, r2_ref, r3_ref, w_ref, uf_ref, w1m_ref, w2m_ref,
              *refs):
    out_ref = refs[-1]
    w = w_ref[0]
    interp = (w[:, 0:1] * r1_ref[...]
              + w[:, 1:2] * r2_ref[...]
              + w[:, 2:3] * r3_ref[...])
    w1a = w1m_ref[:_C2, :]
    w1b = w1m_ref[_C2:, :]
    h = lax.dot_general(w1a, interp, (((0,), (1,)), ((), ())),
                        preferred_element_type=jnp.float32)
    h = h + lax.dot_general(w1b, uf_ref[0], (((0,), (0,)), ((), ())),
                            preferred_element_type=jnp.float32)
    h = jnp.maximum(h, 0.0)
    h = lax.dot_general(w2m_ref[...], h, (((0,), (0,)), ((), ())),
                        preferred_element_type=jnp.float32)
    out_ref[0] = jnp.maximum(h, 0.0)


def _mlp(r1, r2, r3, wts, unknow_feats, W1, W2, prev, b0):
    nb = _N // _NBLK
    in_specs = [
        pl.BlockSpec((_NBLK, _C2), lambda b, n: (b * nb + n, 0)),
        pl.BlockSpec((_NBLK, _C2), lambda b, n: (b * nb + n, 0)),
        pl.BlockSpec((_NBLK, _C2), lambda b, n: (b * nb + n, 0)),
        pl.BlockSpec((1, _NBLK, 3), lambda b, n: (b, n, 0)),
        pl.BlockSpec((1, _C1, _NBLK), lambda b, n: (b + b0, 0, n)),
        pl.BlockSpec((_C1 + _C2, 256), lambda b, n: (0, 0)),
        pl.BlockSpec((256, 256), lambda b, n: (0, 0)),
    ]
    args = [r1, r2, r3, wts, unknow_feats, W1, W2]
    aliases = {}
    if prev is not None:
        in_specs.append(pl.BlockSpec(memory_space=pl.ANY))
        args.append(prev)
        aliases = {7: 0}
    return pl.pallas_call(
        _mlp_body,
        grid=(_BG, nb),
        in_specs=in_specs,
        out_specs=pl.BlockSpec((1, 256, _NBLK), lambda b, n: (b + b0, 0, n)),
        out_shape=jax.ShapeDtypeStruct((_B, 256, _N), jnp.float32),
        input_output_aliases=aliases,
    )(*args)


@jax.jit
def kernel(unknown, known, unknow_feats, known_feats, W1, W2):
    kt = known.swapaxes(1, 2)

    nn = [_three_nn(unknown, kt, known_feats, g * _BG) for g in range(_G)]
    rows = [_sc_gather(kft2, ip) for (ip, _, kft2) in nn]
    out = None
    for g, (r1, r2, r3) in enumerate(rows):
        out = _mlp(r1, r2, r3, nn[g][1], unknow_feats, W1, W2, out,
                   g * _BG)
    return out


# speedup vs baseline: 1.9703x; 1.0167x over previous
import functools
import jax
import jax.numpy as jnp
from jax import lax
from jax.experimental import pallas as pl
from jax.experimental.pallas import tpu as pltpu
from jax.experimental.pallas import tpu_sc as plsc

_B, _N, _M, _C1, _C2 = 8, 4096, 1024, 128, 256
_NBLK = 2048
_BIG = 3.0e38

_G = 2
_BG = _B // _G
_NW = 32
_PW = _BG * _N // _NW
_CP = 128
_WPB = _N // _PW
_MT = _M // (_N // _NBLK)


def _nn_body(u_ref, kt_ref, kf_ref, ip_ref, w_ref, kft2_ref, *, b0):
    b = pl.program_id(0) + b0
    u = u_ref[0]
    kt = kt_ref[0]

    d2 = jnp.zeros((_NBLK, _M), jnp.float32)
    for d in range(3):
        diff = u[:, d:d + 1] - kt[d:d + 1, :]
        d2 = d2 + diff * diff

    ids = lax.broadcasted_iota(jnp.int32, (_NBLK, _M), 1).astype(jnp.float32)
    cur = d2
    mins = []
    idxs = []
    for _ in range(3):
        m = jnp.min(cur, axis=1, keepdims=True)
        eq = cur == m
        i = jnp.min(jnp.where(eq, ids, jnp.float32(_M)), axis=1,
                    keepdims=True)
        mins.append(m)
        idxs.append(i.astype(jnp.int32))
        cur = jnp.where(eq, _BIG, cur)

    r1 = 1.0 / (mins[0] + 1e-8)
    r2 = 1.0 / (mins[1] + 1e-8)
    r3 = 1.0 / (mins[2] + 1e-8)
    norm = r1 + r2 + r3

    packed = (idxs[0] << 20) + (idxs[1] << 10) + idxs[2]
    ip_ref[0, 0] = jnp.reshape(packed, (_NBLK,))
    w_ref[0] = jnp.concatenate([r1 / norm, r2 / norm, r3 / norm], axis=1)
    kft2_ref[...] = kf_ref[0].T


def _three_nn(unknown, kt, known_feats, b0):
    return pl.pallas_call(
        functools.partial(_nn_body, b0=b0),
        grid=(_BG, _N // _NBLK),
        in_specs=[
            pl.BlockSpec((1, _NBLK, 3), lambda b, n: (b + b0, n, 0)),
            pl.BlockSpec((1, 3, _M), lambda b, n: (b + b0, 0, 0)),
            pl.BlockSpec((1, _C2, _MT), lambda b, n: (b + b0, 0, n)),
        ],
        out_specs=[
            pl.BlockSpec((1, 1, _NBLK), lambda b, n: (b, 0, n)),
            pl.BlockSpec((1, _NBLK, 3), lambda b, n: (b, n, 0)),
            pl.BlockSpec((_MT, _C2), lambda b, n: (b * (_N // _NBLK) + n, 0)),
        ],
        out_shape=[
            jax.ShapeDtypeStruct((_BG, 1, _N), jnp.int32),
            jax.ShapeDtypeStruct((_BG, _N, 3), jnp.float32),
            jax.ShapeDtypeStruct((_BG * _M, _C2), jnp.float32),
        ],
    )(unknown, kt, known_feats)


def _sc_gather_body(kft_hbm, ip_hbm, r1_hbm, r2_hbm, r3_hbm,
                    pk_v, idx_v, rows_v, sem):
    wid = lax.axis_index("s") * 2 + lax.axis_index("c")
    bi = wid // _WPB
    col0 = (wid % _WPB) * _PW

    def chunk_body(ci, carry):
        col = col0 + ci * _CP
        row0 = bi * _N + col
        pltpu.sync_copy(ip_hbm.at[bi, 0, pl.ds(col, _CP)], pk_v)
        base = bi * _M
        for g in range(_CP // 16):
            sl = pl.ds(16 * g, 16)
            pk = pk_v[sl]
            idx_v[0, sl] = (pk >> 20) + base
            idx_v[1, sl] = ((pk >> 10) & 1023) + base
            idx_v[2, sl] = (pk & 1023) + base
        for k, rp in enumerate((r1_hbm, r2_hbm, r3_hbm)):
            pltpu.async_copy(kft_hbm.at[idx_v.at[k]], rows_v.at[k],
                             sem).wait()
            pltpu.sync_copy(rows_v.at[k], rp.at[pl.ds(row0, _CP)])
        return carry

    lax.fori_loop(0, _PW // _CP, chunk_body, 0)


def _sc_gather(kft2, ip):
    mesh = plsc.VectorSubcoreMesh(core_axis_name="c", subcore_axis_name="s")
    rows_t = jax.ShapeDtypeStruct((_BG * _N, _C2), jnp.float32)
    f = functools.partial(
        pl.kernel,
        out_type=[rows_t, rows_t, rows_t],
        mesh=mesh,
        scratch_types=[
            pltpu.VMEM((_CP,), jnp.int32),
            pltpu.VMEM((3, _CP), jnp.int32),
            pltpu.VMEM((3, _CP, _C2), jnp.float32),
            pltpu.SemaphoreType.DMA,
        ],
    )(_sc_gather_body)
    return f(kft2, ip)


def _mlp_body(r1_ref, r2_ref, r3_ref, w_ref, uf_ref, w1m_ref, w2m_ref,
              *refs):
    out_ref = refs[-1]
    w = w_ref[0]
    interp = (w[:, 0:1] * r1_ref[...]
              + w[:, 1:2] * r2_ref[...]
              + w[:, 2:3] * r3_ref[...])
    w1a = w1m_ref[:_C2, :]
    w1b = w1m_ref[_C2:, :]
    h = lax.dot_general(w1a, interp, (((0,), (1,)), ((), ())),
                        preferred_element_type=jnp.float32)
    h = h + lax.dot_general(w1b, uf_ref[0], (((0,), (0,)), ((), ())),
                            preferred_element_type=jnp.float32)
    h = jnp.maximum(h, 0.0)
    h = lax.dot_general(w2m_ref[...], h, (((0,), (0,)), ((), ())),
                        preferred_element_type=jnp.float32)
    out_ref[0] = jnp.maximum(h, 0.0)


def _mlp(r1, r2, r3, wts, unknow_feats, W1, W2, prev, b0):
    nb = _N // _NBLK
    in_specs = [
        pl.BlockSpec((_NBLK, _C2), lambda b, n: (b * nb + n, 0)),
        pl.BlockSpec((_NBLK, _C2), lambda b, n: (b * nb + n, 0)),
        pl.BlockSpec((_NBLK, _C2), lambda b, n: (b * nb + n, 0)),
        pl.BlockSpec((1, _NBLK, 3), lambda b, n: (b, n, 0)),
        pl.BlockSpec((1, _C1, _NBLK), lambda b, n: (b + b0, 0, n)),
        pl.BlockSpec((_C1 + _C2, 256), lambda b, n: (0, 0)),
        pl.BlockSpec((256, 256), lambda b, n: (0, 0)),
    ]
    args = [r1, r2, r3, wts, unknow_feats, W1, W2]
    aliases = {}
    if prev is not None:
        in_specs.append(pl.BlockSpec(memory_space=pl.ANY))
        args.append(prev)
        aliases = {7: 0}
    return pl.pallas_call(
        _mlp_body,
        grid=(_BG, nb),
        in_specs=in_specs,
        out_specs=pl.BlockSpec((1, 256, _NBLK), lambda b, n: (b + b0, 0, n)),
        out_shape=jax.ShapeDtypeStruct((_B, 256, _N), jnp.float32),
        input_output_aliases=aliases,
    )(*args)


@jax.jit
def kernel(unknown, known, unknow_feats, known_feats, W1, W2):
    kt = known.swapaxes(1, 2)

    nn = [_three_nn(unknown, kt, known_feats, g * _BG) for g in range(_G)]
    rows = [_sc_gather(kft2, ip) for (ip, _, kft2) in nn]
    out = None
    for g, (r1, r2, r3) in enumerate(rows):
        out = _mlp(r1, r2, r3, nn[g][1], unknow_feats, W1, W2, out,
                   g * _BG)
    return out


# speedup vs baseline: 2.0224x; 1.0264x over previous
import functools
import jax
import jax.numpy as jnp
from jax import lax
from jax.experimental import pallas as pl
from jax.experimental.pallas import tpu as pltpu
from jax.experimental.pallas import tpu_sc as plsc

_B, _N, _M, _C1, _C2 = 8, 4096, 1024, 128, 256
_NBLK = 2048
_BIG = 3.0e38

_G = 4
_BG = _B // _G
_NW = 32
_PW = _BG * _N // _NW
_CP = 128
_WPB = _N // _PW
_MT = _M // (_N // _NBLK)


def _nn_body(u_ref, kt_ref, kf_ref, ip_ref, w_ref, kft2_ref, *, b0):
    b = pl.program_id(0) + b0
    u = u_ref[0]
    kt = kt_ref[0]

    d2 = jnp.zeros((_NBLK, _M), jnp.float32)
    for d in range(3):
        diff = u[:, d:d + 1] - kt[d:d + 1, :]
        d2 = d2 + diff * diff

    ids = lax.broadcasted_iota(jnp.int32, (_NBLK, _M), 1).astype(jnp.float32)
    cur = d2
    mins = []
    idxs = []
    for _ in range(3):
        m = jnp.min(cur, axis=1, keepdims=True)
        eq = cur == m
        i = jnp.min(jnp.where(eq, ids, jnp.float32(_M)), axis=1,
                    keepdims=True)
        mins.append(m)
        idxs.append(i.astype(jnp.int32))
        cur = jnp.where(eq, _BIG, cur)

    r1 = 1.0 / (mins[0] + 1e-8)
    r2 = 1.0 / (mins[1] + 1e-8)
    r3 = 1.0 / (mins[2] + 1e-8)
    norm = r1 + r2 + r3

    packed = (idxs[0] << 20) + (idxs[1] << 10) + idxs[2]
    ip_ref[0, 0] = jnp.reshape(packed, (_NBLK,))
    w_ref[0] = jnp.concatenate([r1 / norm, r2 / norm, r3 / norm], axis=1)
    kft2_ref[...] = kf_ref[0].T


def _three_nn(unknown, kt, known_feats, b0):
    return pl.pallas_call(
        functools.partial(_nn_body, b0=b0),
        grid=(_BG, _N // _NBLK),
        in_specs=[
            pl.BlockSpec((1, _NBLK, 3), lambda b, n: (b + b0, n, 0)),
            pl.BlockSpec((1, 3, _M), lambda b, n: (b + b0, 0, 0)),
            pl.BlockSpec((1, _C2, _MT), lambda b, n: (b + b0, 0, n)),
        ],
        out_specs=[
            pl.BlockSpec((1, 1, _NBLK), lambda b, n: (b, 0, n)),
            pl.BlockSpec((1, _NBLK, 3), lambda b, n: (b, n, 0)),
            pl.BlockSpec((_MT, _C2), lambda b, n: (b * (_N // _NBLK) + n, 0)),
        ],
        out_shape=[
            jax.ShapeDtypeStruct((_BG, 1, _N), jnp.int32),
            jax.ShapeDtypeStruct((_BG, _N, 3), jnp.float32),
            jax.ShapeDtypeStruct((_BG * _M, _C2), jnp.float32),
        ],
    )(unknown, kt, known_feats)


def _sc_gather_body(kft_hbm, ip_hbm, r1_hbm, r2_hbm, r3_hbm,
                    pk_v, idx_v, rows_v, sem):
    wid = lax.axis_index("s") * 2 + lax.axis_index("c")
    bi = wid // _WPB
    col0 = (wid % _WPB) * _PW

    def chunk_body(ci, carry):
        col = col0 + ci * _CP
        row0 = bi * _N + col
        pltpu.sync_copy(ip_hbm.at[bi, 0, pl.ds(col, _CP)], pk_v)
        base = bi * _M
        for g in range(_CP // 16):
            sl = pl.ds(16 * g, 16)
            pk = pk_v[sl]
            idx_v[0, sl] = (pk >> 20) + base
            idx_v[1, sl] = ((pk >> 10) & 1023) + base
            idx_v[2, sl] = (pk & 1023) + base
        for k, rp in enumerate((r1_hbm, r2_hbm, r3_hbm)):
            pltpu.async_copy(kft_hbm.at[idx_v.at[k]], rows_v.at[k],
                             sem).wait()
            pltpu.sync_copy(rows_v.at[k], rp.at[pl.ds(row0, _CP)])
        return carry

    lax.fori_loop(0, _PW // _CP, chunk_body, 0)


def _sc_gather(kft2, ip):
    mesh = plsc.VectorSubcoreMesh(core_axis_name="c", subcore_axis_name="s")
    rows_t = jax.ShapeDtypeStruct((_BG * _N, _C2), jnp.float32)
    f = functools.partial(
        pl.kernel,
        out_type=[rows_t, rows_t, rows_t],
        mesh=mesh,
        scratch_types=[
            pltpu.VMEM((_CP,), jnp.int32),
            pltpu.VMEM((3, _CP), jnp.int32),
            pltpu.VMEM((3, _CP, _C2), jnp.float32),
            pltpu.SemaphoreType.DMA,
        ],
    )(_sc_gather_body)
    return f(kft2, ip)


def _mlp_body(r1_ref, r2_ref, r3_ref, w_ref, uf_ref, w1m_ref, w2m_ref,
              *refs):
    out_ref = refs[-1]
    w = w_ref[0]
    interp = (w[:, 0:1] * r1_ref[...]
              + w[:, 1:2] * r2_ref[...]
              + w[:, 2:3] * r3_ref[...])
    w1a = w1m_ref[:_C2, :]
    w1b = w1m_ref[_C2:, :]
    h = lax.dot_general(w1a, interp, (((0,), (1,)), ((), ())),
                        preferred_element_type=jnp.float32)
    h = h + lax.dot_general(w1b, uf_ref[0], (((0,), (0,)), ((), ())),
                            preferred_element_type=jnp.float32)
    h = jnp.maximum(h, 0.0)
    h = lax.dot_general(w2m_ref[...], h, (((0,), (0,)), ((), ())),
                        preferred_element_type=jnp.float32)
    out_ref[0] = jnp.maximum(h, 0.0)


def _mlp(r1, r2, r3, wts, unknow_feats, W1, W2, prev, b0):
    nb = _N // _NBLK
    in_specs = [
        pl.BlockSpec((_NBLK, _C2), lambda b, n: (b * nb + n, 0)),
        pl.BlockSpec((_NBLK, _C2), lambda b, n: (b * nb + n, 0)),
        pl.BlockSpec((_NBLK, _C2), lambda b, n: (b * nb + n, 0)),
        pl.BlockSpec((1, _NBLK, 3), lambda b, n: (b, n, 0)),
        pl.BlockSpec((1, _C1, _NBLK), lambda b, n: (b + b0, 0, n)),
        pl.BlockSpec((_C1 + _C2, 256), lambda b, n: (0, 0)),
        pl.BlockSpec((256, 256), lambda b, n: (0, 0)),
    ]
    args = [r1, r2, r3, wts, unknow_feats, W1, W2]
    aliases = {}
    if prev is not None:
        in_specs.append(pl.BlockSpec(memory_space=pl.ANY))
        args.append(prev)
        aliases = {7: 0}
    return pl.pallas_call(
        _mlp_body,
        grid=(_BG, nb),
        in_specs=in_specs,
        out_specs=pl.BlockSpec((1, 256, _NBLK), lambda b, n: (b + b0, 0, n)),
        out_shape=jax.ShapeDtypeStruct((_B, 256, _N), jnp.float32),
        input_output_aliases=aliases,
    )(*args)


@jax.jit
def kernel(unknown, known, unknow_feats, known_feats, W1, W2):
    kt = known.swapaxes(1, 2)

    nn = [_three_nn(unknown, kt, known_feats, g * _BG) for g in range(_G)]
    rows = [_sc_gather(kft2, ip) for (ip, _, kft2) in nn]
    out = None
    for g, (r1, r2, r3) in enumerate(rows):
        out = _mlp(r1, r2, r3, nn[g][1], unknow_feats, W1, W2, out,
                   g * _BG)
    return out
